# baseline (speedup 1.0000x reference)
import functools

import jax
import jax.numpy as jnp
from jax import lax
from jax.experimental import pallas as pl
from jax.experimental.pallas import tpu as pltpu
from jax.experimental.pallas import tpu_sc as plsc

B, S, D, H, E, K = 1, 2048, 768, 3072, 8, 2
T = B * S
TK = T * K

BLK = 128
NB = TK // BLK + E
NPAD = NB * BLK
HC = 512
NH = H // HC

NC, NS = 2, 16
NW = NC * NS


def _router_body(x_ref, rw_ref, idx_ref, w_ref):
    x = x_ref[...]
    rw = rw_ref[...]
    logits = lax.dot_general(x, rw, (((1,), (1,)), ((), ())),
                             preferred_element_type=jnp.float32)
    iota = lax.broadcasted_iota(jnp.int32, logits.shape, 1)
    m1 = jnp.max(logits, axis=1, keepdims=True)
    i1 = jnp.min(jnp.where(logits == m1, iota, E), axis=1, keepdims=True)
    l2 = jnp.where(iota == i1, -jnp.inf, logits)
    m2 = jnp.max(l2, axis=1, keepdims=True)
    i2 = jnp.min(jnp.where(l2 == m2, iota, E), axis=1, keepdims=True)
    e = jnp.exp(m2 - m1)
    w1 = 1.0 / (1.0 + e)
    idx_ref[...] = jnp.concatenate([i1, i2], axis=1)
    w_ref[...] = jnp.concatenate([w1, e * w1], axis=1)


def _router(flat, router_w):
    return pl.pallas_call(
        _router_body,
        out_shape=(jax.ShapeDtypeStruct((T, K), jnp.int32),
                   jax.ShapeDtypeStruct((T, K), jnp.float32)),
    )(flat, router_w)


_RPW = NPAD // NW
_GCH = 80


def _sc_mesh():
    return plsc.VectorSubcoreMesh(core_axis_name="c", subcore_axis_name="s",
                                  num_cores=NC, num_subcores=NS)


def _sc_gather(src_tok, flat):
    @functools.partial(
        pl.kernel,
        out_type=jax.ShapeDtypeStruct((NPAD, D), jnp.float32),
        mesh=_sc_mesh(),
        scratch_types=[pltpu.VMEM((_GCH,), jnp.int32),
                       pltpu.VMEM((_GCH, D), jnp.float32),
                       pltpu.SemaphoreType.DMA],
    )
    def k(tok_hbm, flat_hbm, out_hbm, idx_v, rows_v, sem):
        wid = lax.axis_index("s") * NC + lax.axis_index("c")
        base = wid * _RPW
        for c in range(_RPW // _GCH):
            off = base + c * _GCH
            pltpu.sync_copy(tok_hbm.at[pl.ds(off, _GCH)], idx_v)
            pltpu.async_copy(flat_hbm.at[idx_v], rows_v, sem).wait()
            pltpu.sync_copy(rows_v, out_hbm.at[pl.ds(off, _GCH)])

    return k(src_tok, flat)


def _gmm_body(eid_ref, vld_ref, x_ref, wp_ref, w1_ref, w2_ref, b1_ref, b2_ref,
              out_ref):
    h = pl.program_id(0)
    b = pl.program_id(1)
    nh = pl.num_programs(0)
    e = eid_ref[b]

    @pl.when(vld_ref[b] == 1)
    def _():
        rows = pl.ds(b * BLK, BLK)
        x = x_ref[rows, :].astype(jnp.bfloat16)
        acc = lax.dot_general(x, w1_ref[0], (((1,), (1,)), ((), ())),
                              preferred_element_type=jnp.float32)
        acc = acc + b1_ref[pl.ds(e, 1), pl.ds(h * HC, HC)]
        g = jax.nn.gelu(acc, approximate=True).astype(jnp.bfloat16)
        y = lax.dot_general(g, w2_ref[0], (((1,), (1,)), ((), ())),
                            preferred_element_type=jnp.float32)

        @pl.when(h == 0)
        def _():
            out_ref[rows, :] = y

        @pl.when(jnp.logical_and(h > 0, h < nh - 1))
        def _():
            out_ref[rows, :] = out_ref[rows, :] + y

        @pl.when(h == nh - 1)
        def _():
            b2 = b2_ref[pl.ds(e, 1), :]
            wcol = wp_ref[rows, :]
            out_ref[rows, :] = (out_ref[rows, :] + y + b2) * wcol


def _gmm(eid, vld, x_pad, w_pad, fc1_w, fc2_w, fc1_b, fc2_b):
    grid_spec = pltpu.PrefetchScalarGridSpec(
        num_scalar_prefetch=2,
        grid=(NH, NB),
        in_specs=[
            pl.BlockSpec((NPAD, D), lambda h, b, eid, vld: (0, 0)),
            pl.BlockSpec((NPAD, 1), lambda h, b, eid, vld: (0, 0)),
            pl.BlockSpec((1, HC, D), lambda h, b, eid, vld: (eid[b], h, 0)),
            pl.BlockSpec((1, D, HC), lambda h, b, eid, vld: (eid[b], 0, h)),
            pl.BlockSpec((E, H), lambda h, b, eid, vld: (0, 0)),
            pl.BlockSpec((E, D), lambda h, b, eid, vld: (0, 0)),
        ],
        out_specs=pl.BlockSpec((NPAD, D), lambda h, b, eid, vld: (0, 0)),
    )
    return pl.pallas_call(
        _gmm_body,
        grid_spec=grid_spec,
        out_shape=jax.ShapeDtypeStruct((NPAD, D), jnp.float32),
    )(eid, vld, x_pad, w_pad, fc1_w, fc2_w, fc1_b, fc2_b)


_TPW = T // NW
_NVEC = D // 16


def _sc_combine(pos0, pos1, y_pad):
    @functools.partial(
        pl.kernel,
        out_type=jax.ShapeDtypeStruct((T, D), jnp.float32),
        mesh=_sc_mesh(),
        scratch_types=[pltpu.VMEM((_TPW,), jnp.int32),
                       pltpu.VMEM((_TPW,), jnp.int32),
                       pltpu.VMEM((_TPW, D), jnp.float32),
                       pltpu.VMEM((_TPW, D), jnp.float32),
                       pltpu.SemaphoreType.DMA],
    )
    def k(pos0_hbm, pos1_hbm, y_hbm, out_hbm, i0_v, i1_v, a_v, b_v, sem):
        wid = lax.axis_index("s") * NC + lax.axis_index("c")
        base = wid * _TPW
        pltpu.sync_copy(pos0_hbm.at[pl.ds(base, _TPW)], i0_v)
        pltpu.sync_copy(pos1_hbm.at[pl.ds(base, _TPW)], i1_v)
        cp0 = pltpu.async_copy(y_hbm.at[i0_v], a_v, sem)
        cp1 = pltpu.async_copy(y_hbm.at[i1_v], b_v, sem)
        cp0.wait()
        cp1.wait()

        def body(i, carry):
            r = i // _NVEC
            c = (i % _NVEC) * 16
            a_v[r, pl.ds(c, 16)] = a_v[r, pl.ds(c, 16)] + b_v[r, pl.ds(c, 16)]
            return carry

        lax.fori_loop(0, _TPW * _NVEC, body, 0)
        pltpu.sync_copy(a_v, out_hbm.at[pl.ds(base, _TPW)])

    return k(pos0, pos1, y_pad)


def kernel(x, router_w, fc1_w, fc1_b, fc2_w, fc2_b):
    flat = x.reshape(T, D)
    idx, w = _router(flat, router_w)

    e_all = idx.reshape(TK)
    oh = (e_all[:, None] == jnp.arange(E, dtype=jnp.int32)[None, :])
    oh = oh.astype(jnp.int32)
    csum = jnp.cumsum(oh, axis=0)
    counts = csum[-1]
    rank = jnp.sum(oh * (csum - 1), axis=1)
    nb = (counts + BLK - 1) // BLK
    cum_nb = jnp.cumsum(nb)
    pstart = (cum_nb - nb) * BLK
    dest = jnp.take(pstart, e_all) + rank

    src_tok = jnp.zeros(NPAD, jnp.int32).at[dest].set(
        jnp.arange(TK, dtype=jnp.int32) // K)
    w_pad = jnp.zeros((NPAD, 1), jnp.float32).at[dest, 0].set(w.reshape(TK))
    barange = jnp.arange(NB, dtype=jnp.int32)
    eid = jnp.minimum(
        jnp.sum((barange[:, None] >= cum_nb[None, :]).astype(jnp.int32),
                axis=1), E - 1).astype(jnp.int32)
    vld = (barange < cum_nb[E - 1]).astype(jnp.int32)

    x_pad = _sc_gather(src_tok, flat)
    y_pad = _gmm(eid, vld, x_pad, w_pad,
                 fc1_w.astype(jnp.bfloat16), fc2_w.astype(jnp.bfloat16),
                 fc1_b, fc2_b)
    pos = dest.reshape(T, K)
    out = _sc_combine(pos[:, 0], pos[:, 1], y_pad)
    return out.reshape(B, S, D)


# speedup vs baseline: 1.3793x; 1.3793x over previous
import functools

import jax
import jax.numpy as jnp
from jax import lax
from jax.experimental import pallas as pl
from jax.experimental.pallas import tpu as pltpu
from jax.experimental.pallas import tpu_sc as plsc

B, S, D, H, E, K = 1, 2048, 768, 3072, 8, 2
T = B * S
TK = T * K

BLK = 128
NB = TK // BLK + E
NPAD = NB * BLK
HC = 512
NH = H // HC
PG = 256
NG = T // PG

NC, NS = 2, 16
NW = NC * NS


def _route_body(x_ref, rw_ref, d0_ref, d1_ref, w0_ref, w1_ref, eid_ref,
                vld_ref):
    x = x_ref[...]
    logits = lax.dot_general(x, rw_ref[...], (((1,), (1,)), ((), ())),
                             preferred_element_type=jnp.float32)
    iota = lax.broadcasted_iota(jnp.int32, (T, E), 1)
    m1 = jnp.max(logits, axis=1, keepdims=True)
    i1 = jnp.min(jnp.where(logits == m1, iota, E), axis=1, keepdims=True)
    l2 = jnp.where(iota == i1, -jnp.inf, logits)
    m2 = jnp.max(l2, axis=1, keepdims=True)
    i2 = jnp.min(jnp.where(l2 == m2, iota, E), axis=1, keepdims=True)
    ew = jnp.exp(m2 - m1)
    w0_ref[...] = 1.0 / (1.0 + ew)
    w1_ref[...] = ew / (1.0 + ew)

    cnt = ((iota == i1).astype(jnp.float32)
           + (iota == i2).astype(jnp.float32))

    gi = lax.broadcasted_iota(jnp.int32, (PG, PG), 0)
    gj = lax.broadcasted_iota(jnp.int32, (PG, PG), 1)
    ltri = (gi > gj).astype(jnp.bfloat16)
    base = jnp.zeros((1, E), jnp.float32)
    prefix_parts = []
    for g in range(NG):
        cg = cnt[g * PG:(g + 1) * PG, :]
        pg = lax.dot_general(ltri, cg.astype(jnp.bfloat16),
                             (((1,), (0,)), ((), ())),
                             preferred_element_type=jnp.float32)
        prefix_parts.append(pg + base)
        base = base + jnp.sum(cg, axis=0, keepdims=True)
    prefix = jnp.concatenate(prefix_parts, axis=0)
    counts = base

    nb = jnp.floor((counts + (BLK - 1)) * (1.0 / BLK))
    ei = lax.broadcasted_iota(jnp.int32, (E, E), 0)
    ej = lax.broadcasted_iota(jnp.int32, (E, E), 1)
    tril8 = (ei <= ej).astype(jnp.float32)
    cum_nb = lax.dot_general(nb, tril8, (((1,), (0,)), ((), ())),
                             preferred_element_type=jnp.float32)
    pstart = (cum_nb - nb) * BLK

    slot = prefix + pstart
    d0 = jnp.sum(jnp.where(iota == i1, slot, 0.0), axis=1, keepdims=True)
    d1 = jnp.sum(jnp.where(iota == i2, slot, 0.0), axis=1, keepdims=True)
    d0_ref[...] = d0.astype(jnp.int32)
    d1_ref[...] = d1.astype(jnp.int32)

    brange = lax.broadcasted_iota(jnp.int32, (1, 128), 1).astype(jnp.float32)
    acc = jnp.zeros((1, 128), jnp.float32)
    for e in range(E):
        ce = cum_nb[:, e:e + 1]
        acc = acc + (brange >= ce).astype(jnp.float32)
    eid_ref[...] = jnp.minimum(acc, E - 1).astype(jnp.int32)
    ctot = cum_nb[:, E - 1:E]
    vld_ref[...] = (brange < ctot).astype(jnp.int32)


def _route(flat, router_w):
    return pl.pallas_call(
        _route_body,
        out_shape=(jax.ShapeDtypeStruct((T, 1), jnp.int32),
                   jax.ShapeDtypeStruct((T, 1), jnp.int32),
                   jax.ShapeDtypeStruct((T, 1), jnp.float32),
                   jax.ShapeDtypeStruct((T, 1), jnp.float32),
                   jax.ShapeDtypeStruct((1, 128), jnp.int32),
                   jax.ShapeDtypeStruct((1, 128), jnp.int32)),
    )(flat, router_w)


_TPW = T // NW


def _sc_mesh():
    return plsc.VectorSubcoreMesh(core_axis_name="c", subcore_axis_name="s",
                                  num_cores=NC, num_subcores=NS)


def _sc_dispatch(pos0, pos1, flat):
    @functools.partial(
        pl.kernel,
        out_type=jax.ShapeDtypeStruct((NPAD, D), jnp.float32),
        mesh=_sc_mesh(),
        scratch_types=[pltpu.VMEM((_TPW,), jnp.int32),
                       pltpu.VMEM((_TPW,), jnp.int32),
                       pltpu.VMEM((_TPW, D), jnp.float32),
                       pltpu.SemaphoreType.DMA],
    )
    def k(pos0_hbm, pos1_hbm, flat_hbm, out_hbm, i0_v, i1_v, rows_v, sem):
        wid = lax.axis_index("s") * NC + lax.axis_index("c")
        base = wid * _TPW
        pltpu.sync_copy(pos0_hbm.at[pl.ds(base, _TPW)], i0_v)
        pltpu.sync_copy(pos1_hbm.at[pl.ds(base, _TPW)], i1_v)
        pltpu.sync_copy(flat_hbm.at[pl.ds(base, _TPW)], rows_v)
        cp0 = pltpu.async_copy(rows_v, out_hbm.at[i0_v], sem)
        cp1 = pltpu.async_copy(rows_v, out_hbm.at[i1_v], sem)
        cp0.wait()
        cp1.wait()

    return k(pos0, pos1, flat)


def _gmm_body(eid_ref, vld_ref, x_ref, wp_ref, w1_ref, w2_ref, b1_ref, b2_ref,
              out_ref):
    h = pl.program_id(0)
    b = pl.program_id(1)
    nh = pl.num_programs(0)
    e = eid_ref[0, b]

    @pl.when(vld_ref[0, b] == 1)
    def _():
        rows = pl.ds(b * BLK, BLK)
        x = x_ref[rows, :]
        acc = lax.dot_general(x, w1_ref[0], (((1,), (1,)), ((), ())),
                              preferred_element_type=jnp.float32)
        acc = acc + b1_ref[pl.ds(e, 1), pl.ds(h * HC, HC)]
        g = jax.nn.gelu(acc, approximate=True)
        y = lax.dot_general(g, w2_ref[0], (((1,), (1,)), ((), ())),
                            preferred_element_type=jnp.float32)

        @pl.when(h == 0)
        def _():
            out_ref[rows, :] = y

        @pl.when(jnp.logical_and(h > 0, h < nh - 1))
        def _():
            out_ref[rows, :] = out_ref[rows, :] + y

        @pl.when(h == nh - 1)
        def _():
            out_ref[rows, :] = ((out_ref[rows, :] + y
                                 + b2_ref[pl.ds(e, 1), :]) * wp_ref[rows, :])


def _gmm(eid, vld, x_pad, w_pad, fc1_w, fc2_w, fc1_b, fc2_b):
    grid_spec = pltpu.PrefetchScalarGridSpec(
        num_scalar_prefetch=2,
        grid=(NH, NB),
        in_specs=[
            pl.BlockSpec((NPAD, D), lambda h, b, eid, vld: (0, 0)),
            pl.BlockSpec((NPAD, 1), lambda h, b, eid, vld: (0, 0)),
            pl.BlockSpec((1, HC, D), lambda h, b, eid, vld: (eid[0, b], h, 0)),
            pl.BlockSpec((1, D, HC), lambda h, b, eid, vld: (eid[0, b], 0, h)),
            pl.BlockSpec((E, H), lambda h, b, eid, vld: (0, 0)),
            pl.BlockSpec((E, D), lambda h, b, eid, vld: (0, 0)),
        ],
        out_specs=pl.BlockSpec((NPAD, D), lambda h, b, eid, vld: (0, 0)),
    )
    return pl.pallas_call(
        _gmm_body,
        grid_spec=grid_spec,
        out_shape=jax.ShapeDtypeStruct((NPAD, D), jnp.float32),
    )(eid, vld, x_pad, w_pad, fc1_w, fc2_w, fc1_b, fc2_b)


_NVEC = D // 16


def _sc_combine(pos0, pos1, y_pad):
    @functools.partial(
        pl.kernel,
        out_type=jax.ShapeDtypeStruct((T, D), jnp.float32),
        mesh=_sc_mesh(),
        scratch_types=[pltpu.VMEM((_TPW,), jnp.int32),
                       pltpu.VMEM((_TPW,), jnp.int32),
                       pltpu.VMEM((_TPW, D), jnp.float32),
                       pltpu.VMEM((_TPW, D), jnp.float32),
                       pltpu.SemaphoreType.DMA],
    )
    def k(pos0_hbm, pos1_hbm, y_hbm, out_hbm, i0_v, i1_v, a_v, b_v, sem):
        wid = lax.axis_index("s") * NC + lax.axis_index("c")
        base = wid * _TPW
        pltpu.sync_copy(pos0_hbm.at[pl.ds(base, _TPW)], i0_v)
        pltpu.sync_copy(pos1_hbm.at[pl.ds(base, _TPW)], i1_v)
        cp0 = pltpu.async_copy(y_hbm.at[i0_v], a_v, sem)
        cp1 = pltpu.async_copy(y_hbm.at[i1_v], b_v, sem)
        cp0.wait()
        cp1.wait()

        def body(i, carry):
            r = i // _NVEC
            cs = pl.ds((i % _NVEC) * 16, 16)
            a_v[r, cs] = a_v[r, cs] + b_v[r, cs]
            return carry

        lax.fori_loop(0, _TPW * _NVEC, body, 0)
        pltpu.sync_copy(a_v, out_hbm.at[pl.ds(base, _TPW)])

    return k(pos0, pos1, y_pad)


def kernel(x, router_w, fc1_w, fc1_b, fc2_w, fc2_b):
    flat = x.reshape(T, D)
    d0, d1, w0, w1, eid, vld = _route(flat, router_w)
    pos0 = d0.reshape(T)
    pos1 = d1.reshape(T)
    w_pad = (jnp.zeros((NPAD, 1), jnp.float32)
             .at[pos0, 0].set(w0.reshape(T))
             .at[pos1, 0].set(w1.reshape(T)))
    x_pad = _sc_dispatch(pos0, pos1, flat)
    y_pad = _gmm(eid, vld, x_pad, w_pad, fc1_w, fc2_w, fc1_b, fc2_b)
    out = _sc_combine(pos0, pos1, y_pad)
    return out.reshape(B, S, D)


# speedup vs baseline: 1.6569x; 1.2013x over previous
import functools

import jax
import jax.numpy as jnp
from jax import lax
from jax.experimental import pallas as pl
from jax.experimental.pallas import tpu as pltpu
from jax.experimental.pallas import tpu_sc as plsc

B, S, D, H, E, K = 1, 2048, 768, 3072, 8, 2
T = B * S
TK = T * K

BLK = 128
NB = TK // BLK + E
NPAD = NB * BLK
HC = 1024
NH = H // HC
PG = 256
NG = T // PG

NC, NS = 2, 16
NW = NC * NS


def _route_body(x_ref, rw_ref, d0_ref, d1_ref, w0_ref, w1_ref, eid_ref,
                vld_ref):
    x = x_ref[...]
    logits = lax.dot_general(x, rw_ref[...], (((1,), (1,)), ((), ())),
                             preferred_element_type=jnp.float32)
    iota = lax.broadcasted_iota(jnp.int32, (T, E), 1)
    m1 = jnp.max(logits, axis=1, keepdims=True)
    i1 = jnp.min(jnp.where(logits == m1, iota, E), axis=1, keepdims=True)
    l2 = jnp.where(iota == i1, -jnp.inf, logits)
    m2 = jnp.max(l2, axis=1, keepdims=True)
    i2 = jnp.min(jnp.where(l2 == m2, iota, E), axis=1, keepdims=True)
    ew = jnp.exp(m2 - m1)
    w0_ref[...] = 1.0 / (1.0 + ew)
    w1_ref[...] = ew / (1.0 + ew)

    cnt = ((iota == i1).astype(jnp.float32)
           + (iota == i2).astype(jnp.float32))

    gi = lax.broadcasted_iota(jnp.int32, (PG, PG), 0)
    gj = lax.broadcasted_iota(jnp.int32, (PG, PG), 1)
    ltri = (gi > gj).astype(jnp.bfloat16)
    base = jnp.zeros((1, E), jnp.float32)
    prefix_parts = []
    for g in range(NG):
        cg = cnt[g * PG:(g + 1) * PG, :]
        pg = lax.dot_general(ltri, cg.astype(jnp.bfloat16),
                             (((1,), (0,)), ((), ())),
                             preferred_element_type=jnp.float32)
        prefix_parts.append(pg + base)
        base = base + jnp.sum(cg, axis=0, keepdims=True)
    prefix = jnp.concatenate(prefix_parts, axis=0)
    counts = base

    nb = jnp.floor((counts + (BLK - 1)) * (1.0 / BLK))
    ei = lax.broadcasted_iota(jnp.int32, (E, E), 0)
    ej = lax.broadcasted_iota(jnp.int32, (E, E), 1)
    tril8 = (ei <= ej).astype(jnp.float32)
    cum_nb = lax.dot_general(nb, tril8, (((1,), (0,)), ((), ())),
                             preferred_element_type=jnp.float32)
    pstart = (cum_nb - nb) * BLK

    slot = prefix + pstart
    d0 = jnp.sum(jnp.where(iota == i1, slot, 0.0), axis=1, keepdims=True)
    d1 = jnp.sum(jnp.where(iota == i2, slot, 0.0), axis=1, keepdims=True)
    d0_ref[...] = d0.astype(jnp.int32)
    d1_ref[...] = d1.astype(jnp.int32)

    brange = lax.broadcasted_iota(jnp.int32, (1, 128), 1).astype(jnp.float32)
    acc = jnp.zeros((1, 128), jnp.float32)
    for e in range(E):
        ce = cum_nb[:, e:e + 1]
        acc = acc + (brange >= ce).astype(jnp.float32)
    eid_ref[...] = jnp.minimum(acc, E - 1).astype(jnp.int32)
    ctot = cum_nb[:, E - 1:E]
    vld_ref[...] = (brange < ctot).astype(jnp.int32)


def _route(flat, router_w):
    return pl.pallas_call(
        _route_body,
        out_shape=(jax.ShapeDtypeStruct((T, 1), jnp.int32),
                   jax.ShapeDtypeStruct((T, 1), jnp.int32),
                   jax.ShapeDtypeStruct((T, 1), jnp.float32),
                   jax.ShapeDtypeStruct((T, 1), jnp.float32),
                   jax.ShapeDtypeStruct((1, 128), jnp.int32),
                   jax.ShapeDtypeStruct((1, 128), jnp.int32)),
    )(flat, router_w)


_TPW = T // NW


def _sc_mesh():
    return plsc.VectorSubcoreMesh(core_axis_name="c", subcore_axis_name="s",
                                  num_cores=NC, num_subcores=NS)


def _sc_dispatch(pos0, pos1, flat):
    @functools.partial(
        pl.kernel,
        out_type=jax.ShapeDtypeStruct((NPAD, D), jnp.float32),
        mesh=_sc_mesh(),
        scratch_types=[pltpu.VMEM((_TPW,), jnp.int32),
                       pltpu.VMEM((_TPW,), jnp.int32),
                       pltpu.VMEM((_TPW, D), jnp.float32),
                       pltpu.SemaphoreType.DMA],
    )
    def k(pos0_hbm, pos1_hbm, flat_hbm, out_hbm, i0_v, i1_v, rows_v, sem):
        wid = lax.axis_index("s") * NC + lax.axis_index("c")
        base = wid * _TPW
        pltpu.sync_copy(pos0_hbm.at[pl.ds(base, _TPW)], i0_v)
        pltpu.sync_copy(pos1_hbm.at[pl.ds(base, _TPW)], i1_v)
        pltpu.sync_copy(flat_hbm.at[pl.ds(base, _TPW)], rows_v)
        cp0 = pltpu.async_copy(rows_v, out_hbm.at[i0_v], sem)
        cp1 = pltpu.async_copy(rows_v, out_hbm.at[i1_v], sem)
        cp0.wait()
        cp1.wait()

    return k(pos0, pos1, flat)


def _gmm_body(eid_ref, vld_ref, x_ref, wp_ref, w1_ref, w2_ref, b1_ref, b2_ref,
              out_ref):
    h = pl.program_id(0)
    b = pl.program_id(1)
    nh = pl.num_programs(0)
    e = eid_ref[0, b]

    @pl.when(vld_ref[0, b] == 1)
    def _():
        rows = pl.ds(b * BLK, BLK)
        x = x_ref[rows, :]
        acc = lax.dot_general(x, w1_ref[0], (((1,), (1,)), ((), ())),
                              preferred_element_type=jnp.float32)
        acc = acc + b1_ref[pl.ds(e, 1), pl.ds(h * HC, HC)]
        g = jax.nn.gelu(acc, approximate=True)
        y = lax.dot_general(g, w2_ref[0], (((1,), (1,)), ((), ())),
                            preferred_element_type=jnp.float32)

        @pl.when(h == 0)
        def _():
            out_ref[rows, :] = y

        @pl.when(jnp.logical_and(h > 0, h < nh - 1))
        def _():
            out_ref[rows, :] = out_ref[rows, :] + y

        @pl.when(h == nh - 1)
        def _():
            out_ref[rows, :] = ((out_ref[rows, :] + y
                                 + b2_ref[pl.ds(e, 1), :]) * wp_ref[rows, :])


def _gmm(eid, vld, x_pad, w_pad, fc1_w, fc2_w, fc1_b, fc2_b):
    grid_spec = pltpu.PrefetchScalarGridSpec(
        num_scalar_prefetch=2,
        grid=(NH, NB),
        in_specs=[
            pl.BlockSpec((NPAD, D), lambda h, b, eid, vld: (0, 0)),
            pl.BlockSpec((NPAD, 1), lambda h, b, eid, vld: (0, 0)),
            pl.BlockSpec((1, HC, D), lambda h, b, eid, vld: (eid[0, b], h, 0)),
            pl.BlockSpec((1, D, HC), lambda h, b, eid, vld: (eid[0, b], 0, h)),
            pl.BlockSpec((E, H), lambda h, b, eid, vld: (0, 0)),
            pl.BlockSpec((E, D), lambda h, b, eid, vld: (0, 0)),
        ],
        out_specs=pl.BlockSpec((NPAD, D), lambda h, b, eid, vld: (0, 0)),
    )
    return pl.pallas_call(
        _gmm_body,
        grid_spec=grid_spec,
        out_shape=jax.ShapeDtypeStruct((NPAD, D), jnp.float32),
    )(eid, vld, x_pad, w_pad, fc1_w, fc2_w, fc1_b, fc2_b)


_NVEC = D // 16


def _sc_combine(pos0, pos1, y_pad):
    @functools.partial(
        pl.kernel,
        out_type=jax.ShapeDtypeStruct((T, D), jnp.float32),
        mesh=_sc_mesh(),
        scratch_types=[pltpu.VMEM((_TPW,), jnp.int32),
                       pltpu.VMEM((_TPW,), jnp.int32),
                       pltpu.VMEM((_TPW, D), jnp.float32),
                       pltpu.VMEM((_TPW, D), jnp.float32),
                       pltpu.SemaphoreType.DMA],
    )
    def k(pos0_hbm, pos1_hbm, y_hbm, out_hbm, i0_v, i1_v, a_v, b_v, sem):
        wid = lax.axis_index("s") * NC + lax.axis_index("c")
        base = wid * _TPW
        pltpu.sync_copy(pos0_hbm.at[pl.ds(base, _TPW)], i0_v)
        pltpu.sync_copy(pos1_hbm.at[pl.ds(base, _TPW)], i1_v)
        cp0 = pltpu.async_copy(y_hbm.at[i0_v], a_v, sem)
        cp1 = pltpu.async_copy(y_hbm.at[i1_v], b_v, sem)
        cp0.wait()
        cp1.wait()

        def body(i, carry):
            r = i // _NVEC
            cs = pl.ds((i % _NVEC) * 16, 16)
            a_v[r, cs] = a_v[r, cs] + b_v[r, cs]
            return carry

        lax.fori_loop(0, _TPW * _NVEC, body, 0)
        pltpu.sync_copy(a_v, out_hbm.at[pl.ds(base, _TPW)])

    return k(pos0, pos1, y_pad)


def kernel(x, router_w, fc1_w, fc1_b, fc2_w, fc2_b):
    flat = x.reshape(T, D)
    d0, d1, w0, w1, eid, vld = _route(flat, router_w)
    pos0 = d0.reshape(T)
    pos1 = d1.reshape(T)
    w_pad = (jnp.zeros((NPAD, 1), jnp.float32)
             .at[pos0, 0].set(w0.reshape(T))
             .at[pos1, 0].set(w1.reshape(T)))
    x_pad = _sc_dispatch(pos0, pos1, flat)
    y_pad = _gmm(eid, vld, x_pad, w_pad, fc1_w, fc2_w, fc1_b, fc2_b)
    out = _sc_combine(pos0, pos1, y_pad)
    return out.reshape(B, S, D)


# speedup vs baseline: 2.1846x; 1.3184x over previous
import functools

import jax
import jax.numpy as jnp
from jax import lax
from jax.experimental import pallas as pl
from jax.experimental.pallas import tpu as pltpu
from jax.experimental.pallas import tpu_sc as plsc

B, S, D, H, E, K = 1, 2048, 768, 3072, 8, 2
T = B * S
TK = T * K

BLK = 256
NB = TK // BLK + E
NPAD = NB * BLK
HC = 1024
NH = H // HC
PG = 256
NG = T // PG

NC, NS = 2, 16
NW = NC * NS


def _route_body(x_ref, rw_ref, d0_ref, d1_ref, w0_ref, w1_ref, eid_ref,
                vld_ref):
    x = x_ref[...]
    logits = lax.dot_general(x, rw_ref[...], (((1,), (1,)), ((), ())),
                             preferred_element_type=jnp.float32)
    iota = lax.broadcasted_iota(jnp.int32, (T, E), 1)
    m1 = jnp.max(logits, axis=1, keepdims=True)
    i1 = jnp.min(jnp.where(logits == m1, iota, E), axis=1, keepdims=True)
    l2 = jnp.where(iota == i1, -jnp.inf, logits)
    m2 = jnp.max(l2, axis=1, keepdims=True)
    i2 = jnp.min(jnp.where(l2 == m2, iota, E), axis=1, keepdims=True)
    ew = jnp.exp(m2 - m1)
    w0_ref[...] = 1.0 / (1.0 + ew)
    w1_ref[...] = ew / (1.0 + ew)

    cnt = ((iota == i1).astype(jnp.float32)
           + (iota == i2).astype(jnp.float32))

    gi = lax.broadcasted_iota(jnp.int32, (PG, PG), 0)
    gj = lax.broadcasted_iota(jnp.int32, (PG, PG), 1)
    ltri = (gi > gj).astype(jnp.bfloat16)
    base = jnp.zeros((1, E), jnp.float32)
    prefix_parts = []
    for g in range(NG):
        cg = cnt[g * PG:(g + 1) * PG, :]
        pg = lax.dot_general(ltri, cg.astype(jnp.bfloat16),
                             (((1,), (0,)), ((), ())),
                             preferred_element_type=jnp.float32)
        prefix_parts.append(pg + base)
        base = base + jnp.sum(cg, axis=0, keepdims=True)
    prefix = jnp.concatenate(prefix_parts, axis=0)
    counts = base

    nb = jnp.floor((counts + (BLK - 1)) * (1.0 / BLK))
    ei = lax.broadcasted_iota(jnp.int32, (E, E), 0)
    ej = lax.broadcasted_iota(jnp.int32, (E, E), 1)
    tril8 = (ei <= ej).astype(jnp.float32)
    cum_nb = lax.dot_general(nb, tril8, (((1,), (0,)), ((), ())),
                             preferred_element_type=jnp.float32)
    pstart = (cum_nb - nb) * BLK

    slot = prefix + pstart
    d0 = jnp.sum(jnp.where(iota == i1, slot, 0.0), axis=1, keepdims=True)
    d1 = jnp.sum(jnp.where(iota == i2, slot, 0.0), axis=1, keepdims=True)
    d0_ref[...] = d0.astype(jnp.int32)
    d1_ref[...] = d1.astype(jnp.int32)

    brange = lax.broadcasted_iota(jnp.int32, (1, 128), 1).astype(jnp.float32)
    acc = jnp.zeros((1, 128), jnp.float32)
    for e in range(E):
        ce = cum_nb[:, e:e + 1]
        acc = acc + (brange >= ce).astype(jnp.float32)
    eid_ref[...] = jnp.minimum(acc, E - 1).astype(jnp.int32)
    ctot = cum_nb[:, E - 1:E]
    vld_ref[...] = (brange < ctot).astype(jnp.int32)


def _route(flat, router_w):
    return pl.pallas_call(
        _route_body,
        out_shape=(jax.ShapeDtypeStruct((T, 1), jnp.int32),
                   jax.ShapeDtypeStruct((T, 1), jnp.int32),
                   jax.ShapeDtypeStruct((T, 1), jnp.float32),
                   jax.ShapeDtypeStruct((T, 1), jnp.float32),
                   jax.ShapeDtypeStruct((1, 128), jnp.int32),
                   jax.ShapeDtypeStruct((1, 128), jnp.int32)),
    )(flat, router_w)


_TPW = T // NW


def _sc_mesh():
    return plsc.VectorSubcoreMesh(core_axis_name="c", subcore_axis_name="s",
                                  num_cores=NC, num_subcores=NS)


def _sc_dispatch(pos0, pos1, flat):
    @functools.partial(
        pl.kernel,
        out_type=jax.ShapeDtypeStruct((NPAD, D), jnp.float32),
        mesh=_sc_mesh(),
        scratch_types=[pltpu.VMEM((_TPW,), jnp.int32),
                       pltpu.VMEM((_TPW,), jnp.int32),
                       pltpu.VMEM((_TPW, D), jnp.float32),
                       pltpu.SemaphoreType.DMA],
    )
    def k(pos0_hbm, pos1_hbm, flat_hbm, out_hbm, i0_v, i1_v, rows_v, sem):
        wid = lax.axis_index("s") * NC + lax.axis_index("c")
        base = wid * _TPW
        pltpu.sync_copy(pos0_hbm.at[pl.ds(base, _TPW)], i0_v)
        pltpu.sync_copy(pos1_hbm.at[pl.ds(base, _TPW)], i1_v)
        pltpu.sync_copy(flat_hbm.at[pl.ds(base, _TPW)], rows_v)
        cp0 = pltpu.async_copy(rows_v, out_hbm.at[i0_v], sem)
        cp1 = pltpu.async_copy(rows_v, out_hbm.at[i1_v], sem)
        cp0.wait()
        cp1.wait()

    return k(pos0, pos1, flat)


def _gmm_body(eid_ref, vld_ref, x_ref, wp_ref, w1_ref, w2_ref, b1_ref, b2_ref,
              out_ref):
    h = pl.program_id(0)
    b = pl.program_id(1)
    nh = pl.num_programs(0)
    e = eid_ref[0, b]

    @pl.when(vld_ref[0, b] == 1)
    def _():
        rows = pl.ds(b * BLK, BLK)
        x = x_ref[rows, :]
        acc = lax.dot_general(x, w1_ref[0], (((1,), (1,)), ((), ())),
                              preferred_element_type=jnp.float32)
        acc = acc + b1_ref[pl.ds(e, 1), pl.ds(h * HC, HC)]
        g = jax.nn.gelu(acc, approximate=True)
        y = lax.dot_general(g, w2_ref[0], (((1,), (1,)), ((), ())),
                            preferred_element_type=jnp.float32)

        @pl.when(h == 0)
        def _():
            out_ref[rows, :] = y

        @pl.when(jnp.logical_and(h > 0, h < nh - 1))
        def _():
            out_ref[rows, :] = out_ref[rows, :] + y

        @pl.when(h == nh - 1)
        def _():
            out_ref[rows, :] = ((out_ref[rows, :] + y
                                 + b2_ref[pl.ds(e, 1), :]) * wp_ref[rows, :])


def _gmm(eid, vld, x_pad, w_pad, fc1_w, fc2_w, fc1_b, fc2_b):
    grid_spec = pltpu.PrefetchScalarGridSpec(
        num_scalar_prefetch=2,
        grid=(NH, NB),
        in_specs=[
            pl.BlockSpec((NPAD, D), lambda h, b, eid, vld: (0, 0)),
            pl.BlockSpec((NPAD, 1), lambda h, b, eid, vld: (0, 0)),
            pl.BlockSpec((1, HC, D), lambda h, b, eid, vld: (eid[0, b], h, 0)),
            pl.BlockSpec((1, D, HC), lambda h, b, eid, vld: (eid[0, b], 0, h)),
            pl.BlockSpec((E, H), lambda h, b, eid, vld: (0, 0)),
            pl.BlockSpec((E, D), lambda h, b, eid, vld: (0, 0)),
        ],
        out_specs=pl.BlockSpec((NPAD, D), lambda h, b, eid, vld: (0, 0)),
    )
    return pl.pallas_call(
        _gmm_body,
        grid_spec=grid_spec,
        out_shape=jax.ShapeDtypeStruct((NPAD, D), jnp.float32),
    )(eid, vld, x_pad, w_pad, fc1_w, fc2_w, fc1_b, fc2_b)


_NVEC = D // 16


def _sc_combine(pos0, pos1, y_pad):
    @functools.partial(
        pl.kernel,
        out_type=jax.ShapeDtypeStruct((T, D), jnp.float32),
        mesh=_sc_mesh(),
        scratch_types=[pltpu.VMEM((_TPW,), jnp.int32),
                       pltpu.VMEM((_TPW,), jnp.int32),
                       pltpu.VMEM((_TPW, D), jnp.float32),
                       pltpu.VMEM((_TPW, D), jnp.float32),
                       pltpu.SemaphoreType.DMA],
    )
    def k(pos0_hbm, pos1_hbm, y_hbm, out_hbm, i0_v, i1_v, a_v, b_v, sem):
        wid = lax.axis_index("s") * NC + lax.axis_index("c")
        base = wid * _TPW
        pltpu.sync_copy(pos0_hbm.at[pl.ds(base, _TPW)], i0_v)
        pltpu.sync_copy(pos1_hbm.at[pl.ds(base, _TPW)], i1_v)
        cp0 = pltpu.async_copy(y_hbm.at[i0_v], a_v, sem)
        cp1 = pltpu.async_copy(y_hbm.at[i1_v], b_v, sem)
        cp0.wait()
        cp1.wait()

        def body(i, carry):
            r = i // _NVEC
            cs = pl.ds((i % _NVEC) * 16, 16)
            a_v[r, cs] = a_v[r, cs] + b_v[r, cs]
            return carry

        lax.fori_loop(0, _TPW * _NVEC, body, 0)
        pltpu.sync_copy(a_v, out_hbm.at[pl.ds(base, _TPW)])

    return k(pos0, pos1, y_pad)


def kernel(x, router_w, fc1_w, fc1_b, fc2_w, fc2_b):
    flat = x.reshape(T, D)
    d0, d1, w0, w1, eid, vld = _route(flat, router_w)
    pos0 = d0.reshape(T)
    pos1 = d1.reshape(T)
    w_pad = (jnp.zeros((NPAD, 1), jnp.float32)
             .at[pos0, 0].set(w0.reshape(T))
             .at[pos1, 0].set(w1.reshape(T)))
    x_pad = _sc_dispatch(pos0, pos1, flat)
    y_pad = _gmm(eid, vld, x_pad, w_pad, fc1_w, fc2_w, fc1_b, fc2_b)
    out = _sc_combine(pos0, pos1, y_pad)
    return out.reshape(B, S, D)


# speedup vs baseline: 2.3121x; 1.0584x over previous
import functools

import jax
import jax.numpy as jnp
from jax import lax
from jax.experimental import pallas as pl
from jax.experimental.pallas import tpu as pltpu
from jax.experimental.pallas import tpu_sc as plsc

B, S, D, H, E, K = 1, 2048, 768, 3072, 8, 2
T = B * S
TK = T * K

BLK = 256
NB = TK // BLK + E
NPAD = NB * BLK
HC = 1536
NH = H // HC
PG = 256
NG = T // PG

NC, NS = 2, 16
NW = NC * NS


def _route_body(x_ref, rw_ref, d0_ref, d1_ref, w0_ref, w1_ref, eid_ref,
                vld_ref):
    x = x_ref[...]
    logits = lax.dot_general(x, rw_ref[...], (((1,), (1,)), ((), ())),
                             preferred_element_type=jnp.float32)
    iota = lax.broadcasted_iota(jnp.int32, (T, E), 1)
    m1 = jnp.max(logits, axis=1, keepdims=True)
    i1 = jnp.min(jnp.where(logits == m1, iota, E), axis=1, keepdims=True)
    l2 = jnp.where(iota == i1, -jnp.inf, logits)
    m2 = jnp.max(l2, axis=1, keepdims=True)
    i2 = jnp.min(jnp.where(l2 == m2, iota, E), axis=1, keepdims=True)
    ew = jnp.exp(m2 - m1)
    w0_ref[...] = 1.0 / (1.0 + ew)
    w1_ref[...] = ew / (1.0 + ew)

    cnt = ((iota == i1).astype(jnp.float32)
           + (iota == i2).astype(jnp.float32))

    gi = lax.broadcasted_iota(jnp.int32, (PG, PG), 0)
    gj = lax.broadcasted_iota(jnp.int32, (PG, PG), 1)
    ltri = (gi > gj).astype(jnp.bfloat16)
    base = jnp.zeros((1, E), jnp.float32)
    prefix_parts = []
    for g in range(NG):
        cg = cnt[g * PG:(g + 1) * PG, :]
        pg = lax.dot_general(ltri, cg.astype(jnp.bfloat16),
                             (((1,), (0,)), ((), ())),
                             preferred_element_type=jnp.float32)
        prefix_parts.append(pg + base)
        base = base + jnp.sum(cg, axis=0, keepdims=True)
    prefix = jnp.concatenate(prefix_parts, axis=0)
    counts = base

    nb = jnp.floor((counts + (BLK - 1)) * (1.0 / BLK))
    ei = lax.broadcasted_iota(jnp.int32, (E, E), 0)
    ej = lax.broadcasted_iota(jnp.int32, (E, E), 1)
    tril8 = (ei <= ej).astype(jnp.float32)
    cum_nb = lax.dot_general(nb, tril8, (((1,), (0,)), ((), ())),
                             preferred_element_type=jnp.float32)
    pstart = (cum_nb - nb) * BLK

    slot = prefix + pstart
    d0 = jnp.sum(jnp.where(iota == i1, slot, 0.0), axis=1, keepdims=True)
    d1 = jnp.sum(jnp.where(iota == i2, slot, 0.0), axis=1, keepdims=True)
    d0_ref[...] = d0.astype(jnp.int32)
    d1_ref[...] = d1.astype(jnp.int32)

    brange = lax.broadcasted_iota(jnp.int32, (1, 128), 1).astype(jnp.float32)
    acc = jnp.zeros((1, 128), jnp.float32)
    for e in range(E):
        ce = cum_nb[:, e:e + 1]
        acc = acc + (brange >= ce).astype(jnp.float32)
    eid_ref[...] = jnp.minimum(acc, E - 1).astype(jnp.int32)
    ctot = cum_nb[:, E - 1:E]
    vld_ref[...] = (brange < ctot).astype(jnp.int32)


def _route(flat, router_w):
    return pl.pallas_call(
        _route_body,
        out_shape=(jax.ShapeDtypeStruct((T, 1), jnp.int32),
                   jax.ShapeDtypeStruct((T, 1), jnp.int32),
                   jax.ShapeDtypeStruct((T, 1), jnp.float32),
                   jax.ShapeDtypeStruct((T, 1), jnp.float32),
                   jax.ShapeDtypeStruct((1, 128), jnp.int32),
                   jax.ShapeDtypeStruct((1, 128), jnp.int32)),
    )(flat, router_w)


_TPW = T // NW


def _sc_mesh():
    return plsc.VectorSubcoreMesh(core_axis_name="c", subcore_axis_name="s",
                                  num_cores=NC, num_subcores=NS)


def _sc_dispatch(pos0, pos1, flat):
    @functools.partial(
        pl.kernel,
        out_type=jax.ShapeDtypeStruct((NPAD, D), jnp.float32),
        mesh=_sc_mesh(),
        scratch_types=[pltpu.VMEM((_TPW,), jnp.int32),
                       pltpu.VMEM((_TPW,), jnp.int32),
                       pltpu.VMEM((_TPW, D), jnp.float32),
                       pltpu.SemaphoreType.DMA],
    )
    def k(pos0_hbm, pos1_hbm, flat_hbm, out_hbm, i0_v, i1_v, rows_v, sem):
        wid = lax.axis_index("s") * NC + lax.axis_index("c")
        base = wid * _TPW
        pltpu.sync_copy(pos0_hbm.at[pl.ds(base, _TPW)], i0_v)
        pltpu.sync_copy(pos1_hbm.at[pl.ds(base, _TPW)], i1_v)
        pltpu.sync_copy(flat_hbm.at[pl.ds(base, _TPW)], rows_v)
        cp0 = pltpu.async_copy(rows_v, out_hbm.at[i0_v], sem)
        cp1 = pltpu.async_copy(rows_v, out_hbm.at[i1_v], sem)
        cp0.wait()
        cp1.wait()

    return k(pos0, pos1, flat)


def _gmm_body(eid_ref, vld_ref, x_ref, wp_ref, w1_ref, w2_ref, b1_ref, b2_ref,
              out_ref):
    h = pl.program_id(0)
    b = pl.program_id(1)
    nh = pl.num_programs(0)
    e = eid_ref[0, b]

    @pl.when(vld_ref[0, b] == 1)
    def _():
        rows = pl.ds(b * BLK, BLK)
        x = x_ref[...]
        acc = lax.dot_general(x, w1_ref[0], (((1,), (1,)), ((), ())),
                              preferred_element_type=jnp.float32)
        acc = acc + b1_ref[pl.ds(e, 1), pl.ds(h * HC, HC)]
        g = jax.nn.gelu(acc, approximate=True)
        y = lax.dot_general(g, w2_ref[0], (((1,), (1,)), ((), ())),
                            preferred_element_type=jnp.float32)

        @pl.when(h == 0)
        def _():
            out_ref[rows, :] = y

        @pl.when(jnp.logical_and(h > 0, h < nh - 1))
        def _():
            out_ref[rows, :] = out_ref[rows, :] + y

        @pl.when(h == nh - 1)
        def _():
            out_ref[rows, :] = ((out_ref[rows, :] + y
                                 + b2_ref[pl.ds(e, 1), :]) * wp_ref[rows, :])


def _gmm(eid, vld, x_pad, w_pad, fc1_w, fc2_w, fc1_b, fc2_b):
    grid_spec = pltpu.PrefetchScalarGridSpec(
        num_scalar_prefetch=2,
        grid=(NH, NB),
        in_specs=[
            pl.BlockSpec((BLK, D), lambda h, b, eid, vld: (b, 0)),
            pl.BlockSpec((NPAD, 1), lambda h, b, eid, vld: (0, 0)),
            pl.BlockSpec((1, HC, D), lambda h, b, eid, vld: (eid[0, b], h, 0)),
            pl.BlockSpec((1, D, HC), lambda h, b, eid, vld: (eid[0, b], 0, h)),
            pl.BlockSpec((E, H), lambda h, b, eid, vld: (0, 0)),
            pl.BlockSpec((E, D), lambda h, b, eid, vld: (0, 0)),
        ],
        out_specs=pl.BlockSpec((NPAD, D), lambda h, b, eid, vld: (0, 0)),
    )
    return pl.pallas_call(
        _gmm_body,
        grid_spec=grid_spec,
        out_shape=jax.ShapeDtypeStruct((NPAD, D), jnp.float32),
    )(eid, vld, x_pad, w_pad, fc1_w, fc2_w, fc1_b, fc2_b)


_NVEC = D // 16


def _sc_combine(pos0, pos1, y_pad):
    @functools.partial(
        pl.kernel,
        out_type=jax.ShapeDtypeStruct((T, D), jnp.float32),
        mesh=_sc_mesh(),
        scratch_types=[pltpu.VMEM((_TPW,), jnp.int32),
                       pltpu.VMEM((_TPW,), jnp.int32),
                       pltpu.VMEM((_TPW, D), jnp.float32),
                       pltpu.VMEM((_TPW, D), jnp.float32),
                       pltpu.SemaphoreType.DMA],
    )
    def k(pos0_hbm, pos1_hbm, y_hbm, out_hbm, i0_v, i1_v, a_v, b_v, sem):
        wid = lax.axis_index("s") * NC + lax.axis_index("c")
        base = wid * _TPW
        pltpu.sync_copy(pos0_hbm.at[pl.ds(base, _TPW)], i0_v)
        pltpu.sync_copy(pos1_hbm.at[pl.ds(base, _TPW)], i1_v)
        cp0 = pltpu.async_copy(y_hbm.at[i0_v], a_v, sem)
        cp1 = pltpu.async_copy(y_hbm.at[i1_v], b_v, sem)
        cp0.wait()
        cp1.wait()

        def body(i, carry):
            r = i // _NVEC
            cs = pl.ds((i % _NVEC) * 16, 16)
            a_v[r, cs] = a_v[r, cs] + b_v[r, cs]
            return carry

        lax.fori_loop(0, _TPW * _NVEC, body, 0)
        pltpu.sync_copy(a_v, out_hbm.at[pl.ds(base, _TPW)])

    return k(pos0, pos1, y_pad)


def kernel(x, router_w, fc1_w, fc1_b, fc2_w, fc2_b):
    flat = x.reshape(T, D)
    d0, d1, w0, w1, eid, vld = _route(flat, router_w)
    pos0 = d0.reshape(T)
    pos1 = d1.reshape(T)
    w_pad = (jnp.zeros((NPAD, 1), jnp.float32)
             .at[pos0, 0].set(w0.reshape(T))
             .at[pos1, 0].set(w1.reshape(T)))
    x_pad = _sc_dispatch(pos0, pos1, flat)
    y_pad = _gmm(eid, vld, x_pad, w_pad, fc1_w, fc2_w, fc1_b, fc2_b)
    out = _sc_combine(pos0, pos1, y_pad)
    return out.reshape(B, S, D)


# speedup vs baseline: 2.6152x; 1.1311x over previous
import functools

import jax
import jax.numpy as jnp
from jax import lax
from jax.experimental import pallas as pl
from jax.experimental.pallas import tpu as pltpu
from jax.experimental.pallas import tpu_sc as plsc

B, S, D, H, E, K = 1, 2048, 768, 3072, 8, 2
T = B * S
TK = T * K

BLK = 256
NB = TK // BLK + E
NPAD = NB * BLK
HC = 1536
NH = H // HC
PG = 256
NG = T // PG

NC, NS = 2, 16
NW = NC * NS


def _route_body(x_ref, rw_ref, d0_ref, d1_ref, w0_ref, w1_ref, eid_ref,
                vld_ref):
    x = x_ref[...]
    logits = lax.dot_general(x, rw_ref[...], (((1,), (1,)), ((), ())),
                             preferred_element_type=jnp.float32)
    iota = lax.broadcasted_iota(jnp.int32, (T, E), 1)
    m1 = jnp.max(logits, axis=1, keepdims=True)
    i1 = jnp.min(jnp.where(logits == m1, iota, E), axis=1, keepdims=True)
    l2 = jnp.where(iota == i1, -jnp.inf, logits)
    m2 = jnp.max(l2, axis=1, keepdims=True)
    i2 = jnp.min(jnp.where(l2 == m2, iota, E), axis=1, keepdims=True)
    ew = jnp.exp(m2 - m1)
    w0_ref[...] = 1.0 / (1.0 + ew)
    w1_ref[...] = ew / (1.0 + ew)

    cnt = ((iota == i1).astype(jnp.float32)
           + (iota == i2).astype(jnp.float32))

    gi = lax.broadcasted_iota(jnp.int32, (PG, PG), 0)
    gj = lax.broadcasted_iota(jnp.int32, (PG, PG), 1)
    ltri = (gi > gj).astype(jnp.bfloat16)
    base = jnp.zeros((1, E), jnp.float32)
    prefix_parts = []
    for g in range(NG):
        cg = cnt[g * PG:(g + 1) * PG, :]
        pg = lax.dot_general(ltri, cg.astype(jnp.bfloat16),
                             (((1,), (0,)), ((), ())),
                             preferred_element_type=jnp.float32)
        prefix_parts.append(pg + base)
        base = base + jnp.sum(cg, axis=0, keepdims=True)
    prefix = jnp.concatenate(prefix_parts, axis=0)
    counts = base

    nb = jnp.floor((counts + (BLK - 1)) * (1.0 / BLK))
    ei = lax.broadcasted_iota(jnp.int32, (E, E), 0)
    ej = lax.broadcasted_iota(jnp.int32, (E, E), 1)
    tril8 = (ei <= ej).astype(jnp.float32)
    cum_nb = lax.dot_general(nb, tril8, (((1,), (0,)), ((), ())),
                             preferred_element_type=jnp.float32)
    pstart = (cum_nb - nb) * BLK

    slot = prefix + pstart
    d0 = jnp.sum(jnp.where(iota == i1, slot, 0.0), axis=1, keepdims=True)
    d1 = jnp.sum(jnp.where(iota == i2, slot, 0.0), axis=1, keepdims=True)
    d0_ref[...] = d0.astype(jnp.int32)
    d1_ref[...] = d1.astype(jnp.int32)

    brange = lax.broadcasted_iota(jnp.int32, (1, 128), 1).astype(jnp.float32)
    acc = jnp.zeros((1, 128), jnp.float32)
    for e in range(E):
        ce = cum_nb[:, e:e + 1]
        acc = acc + (brange >= ce).astype(jnp.float32)
    eid_ref[...] = jnp.minimum(acc, E - 1).astype(jnp.int32)
    ctot = cum_nb[:, E - 1:E]
    vld_ref[...] = (brange < ctot).astype(jnp.int32)


def _route(flat, router_w):
    return pl.pallas_call(
        _route_body,
        out_shape=(jax.ShapeDtypeStruct((T, 1), jnp.int32),
                   jax.ShapeDtypeStruct((T, 1), jnp.int32),
                   jax.ShapeDtypeStruct((T, 1), jnp.float32),
                   jax.ShapeDtypeStruct((T, 1), jnp.float32),
                   jax.ShapeDtypeStruct((1, 128), jnp.int32),
                   jax.ShapeDtypeStruct((1, 128), jnp.int32)),
    )(flat, router_w)


_TPW = T // NW


def _sc_mesh():
    return plsc.VectorSubcoreMesh(core_axis_name="c", subcore_axis_name="s",
                                  num_cores=NC, num_subcores=NS)


def _sc_dispatch(pos0, pos1, flat):
    @functools.partial(
        pl.kernel,
        out_type=jax.ShapeDtypeStruct((NPAD, D), jnp.float32),
        mesh=_sc_mesh(),
        scratch_types=[pltpu.VMEM((_TPW,), jnp.int32),
                       pltpu.VMEM((_TPW,), jnp.int32),
                       pltpu.VMEM((_TPW, D), jnp.float32),
                       pltpu.SemaphoreType.DMA],
    )
    def k(pos0_hbm, pos1_hbm, flat_hbm, out_hbm, i0_v, i1_v, rows_v, sem):
        wid = lax.axis_index("s") * NC + lax.axis_index("c")
        base = wid * _TPW
        pltpu.sync_copy(pos0_hbm.at[pl.ds(base, _TPW)], i0_v)
        pltpu.sync_copy(pos1_hbm.at[pl.ds(base, _TPW)], i1_v)
        pltpu.sync_copy(flat_hbm.at[pl.ds(base, _TPW)], rows_v)
        cp0 = pltpu.async_copy(rows_v, out_hbm.at[i0_v], sem)
        cp1 = pltpu.async_copy(rows_v, out_hbm.at[i1_v], sem)
        cp0.wait()
        cp1.wait()

    return k(pos0, pos1, flat)


def _gmm_body(eid_ref, vld_ref, x_ref, wp_ref, w1_ref, w2_ref, b1_ref, b2_ref,
              out_ref):
    b = pl.program_id(0)
    e = eid_ref[0, b]

    @pl.when(vld_ref[0, b] == 1)
    def _():
        x = x_ref[...]
        acc = lax.dot_general(x, w1_ref[0], (((1,), (1,)), ((), ())),
                              preferred_element_type=jnp.float32)
        acc = acc + b1_ref[pl.ds(e, 1), :]
        g = jax.nn.gelu(acc, approximate=True)
        y = lax.dot_general(g, w2_ref[0], (((1,), (1,)), ((), ())),
                            preferred_element_type=jnp.float32)
        out_ref[...] = ((y + b2_ref[pl.ds(e, 1), :])
                        * wp_ref[pl.ds(b * BLK, BLK), :])


def _gmm(eid, vld, x_pad, w_pad, fc1_w, fc2_w, fc1_b, fc2_b):
    grid_spec = pltpu.PrefetchScalarGridSpec(
        num_scalar_prefetch=2,
        grid=(NB,),
        in_specs=[
            pl.BlockSpec((BLK, D), lambda b, eid, vld: (b, 0)),
            pl.BlockSpec((NPAD, 1), lambda b, eid, vld: (0, 0)),
            pl.BlockSpec((1, H, D), lambda b, eid, vld: (eid[0, b], 0, 0)),
            pl.BlockSpec((1, D, H), lambda b, eid, vld: (eid[0, b], 0, 0)),
            pl.BlockSpec((E, H), lambda b, eid, vld: (0, 0)),
            pl.BlockSpec((E, D), lambda b, eid, vld: (0, 0)),
        ],
        out_specs=pl.BlockSpec((BLK, D), lambda b, eid, vld: (b, 0)),
    )
    return pl.pallas_call(
        _gmm_body,
        grid_spec=grid_spec,
        out_shape=jax.ShapeDtypeStruct((NPAD, D), jnp.float32),
    )(eid, vld, x_pad, w_pad, fc1_w, fc2_w, fc1_b, fc2_b)


_NVEC = D // 16


def _sc_combine(pos0, pos1, y_pad):
    @functools.partial(
        pl.kernel,
        out_type=jax.ShapeDtypeStruct((T, D), jnp.float32),
        mesh=_sc_mesh(),
        scratch_types=[pltpu.VMEM((_TPW,), jnp.int32),
                       pltpu.VMEM((_TPW,), jnp.int32),
                       pltpu.VMEM((_TPW, D), jnp.float32),
                       pltpu.VMEM((_TPW, D), jnp.float32),
                       pltpu.SemaphoreType.DMA],
    )
    def k(pos0_hbm, pos1_hbm, y_hbm, out_hbm, i0_v, i1_v, a_v, b_v, sem):
        wid = lax.axis_index("s") * NC + lax.axis_index("c")
        base = wid * _TPW
        pltpu.sync_copy(pos0_hbm.at[pl.ds(base, _TPW)], i0_v)
        pltpu.sync_copy(pos1_hbm.at[pl.ds(base, _TPW)], i1_v)
        cp0 = pltpu.async_copy(y_hbm.at[i0_v], a_v, sem)
        cp1 = pltpu.async_copy(y_hbm.at[i1_v], b_v, sem)
        cp0.wait()
        cp1.wait()

        def body(i, carry):
            r = i // _NVEC
            cs = pl.ds((i % _NVEC) * 16, 16)
            a_v[r, cs] = a_v[r, cs] + b_v[r, cs]
            return carry

        lax.fori_loop(0, _TPW * _NVEC, body, 0)
        pltpu.sync_copy(a_v, out_hbm.at[pl.ds(base, _TPW)])

    return k(pos0, pos1, y_pad)


def kernel(x, router_w, fc1_w, fc1_b, fc2_w, fc2_b):
    flat = x.reshape(T, D)
    d0, d1, w0, w1, eid, vld = _route(flat, router_w)
    pos0 = d0.reshape(T)
    pos1 = d1.reshape(T)
    w_pad = (jnp.zeros((NPAD, 1), jnp.float32)
             .at[pos0, 0].set(w0.reshape(T))
             .at[pos1, 0].set(w1.reshape(T)))
    x_pad = _sc_dispatch(pos0, pos1, flat)
    y_pad = _gmm(eid, vld, x_pad, w_pad, fc1_w, fc2_w, fc1_b, fc2_b)
    out = _sc_combine(pos0, pos1, y_pad)
    return out.reshape(B, S, D)
